# baseline (speedup 1.0000x reference)
import functools

import jax
import jax.numpy as jnp
from jax import lax
from jax.experimental import pallas as pl
from jax.experimental.pallas import tpu as pltpu
from jax.experimental.pallas import tpu_sc as plsc

K = 1024
D = 768
N = 32768
MOM = 0.9
NC, NS = 2, 16
CG = 8
COLS = D // CG
TH = 2
TOK_PER_W = N // NC // TH
IDXW = 128
BIG = 128
NROUND = TOK_PER_W // BIG


def _rnorm_body(x_ref, o_ref):
    x = x_ref[...]
    ss = jnp.sum(x * x, axis=1)
    o_ref[...] = (1.0 / jnp.maximum(jnp.sqrt(ss), 1e-12)).reshape(o_ref.shape)


def _inv_norms(x):
    blk = 2048
    return pl.pallas_call(
        _rnorm_body,
        out_shape=jax.ShapeDtypeStruct((N // IDXW, IDXW), jnp.float32),
        grid=(N // blk,),
        in_specs=[pl.BlockSpec((blk, D), lambda i: (i, 0))],
        out_specs=pl.BlockSpec((blk // IDXW, IDXW), lambda i: (i, 0)),
    )(x)


_GDN = lax.GatherDimensionNumbers(
    offset_dims=(), collapsed_slice_dims=(0,), start_index_map=(0,))


def _vgather(v, idx16):
    return lax.gather(v, idx16[:, None], _GDN, (1,),
                      mode=lax.GatherScatterMode.PROMISE_IN_BOUNDS)


@functools.partial(
    pl.kernel,
    out_type=[
        jax.ShapeDtypeStruct((TH, K, D), jnp.float32),
        jax.ShapeDtypeStruct((TH, K, D), jnp.float32),
        jax.ShapeDtypeStruct((K, 16), jnp.float32),
        jax.ShapeDtypeStruct((K, 16), jnp.float32),
    ],
    mesh=plsc.VectorSubcoreMesh(core_axis_name="c", subcore_axis_name="s"),
    compiler_params=pltpu.CompilerParams(use_tc_tiling_on_sc=False,
                                         needs_layout_passes=False),
    scratch_types=[
        pltpu.VMEM((2, BIG, COLS), jnp.bfloat16),
        pltpu.VMEM((2, BIG // IDXW, IDXW), jnp.int32),
        pltpu.VMEM((2, BIG // IDXW, IDXW), jnp.float32),
        pltpu.VMEM((K, COLS), jnp.float32),
        pltpu.VMEM((K, 16), jnp.float32),
        pltpu.SemaphoreType.DMA,
        pltpu.SemaphoreType.DMA,
    ],
)
def _sc_segment(p_hbm, lab_hbm, r_hbm, s0_out, s1_out, c0_out, c1_out,
                rows_v, lab_v, r_v, acc_v, cnt_v, sem0, sem1):
    c = lax.axis_index("c")
    s = lax.axis_index("s")
    cg = s % CG
    th = s // CG
    col0 = cg * COLS
    ii = lax.iota(jnp.int32, 16)
    iis = []
    for j in range(COLS // 32):
        iis.append(j * 32 + 2 * ii)
        iis.append(j * 32 + 2 * ii + 1)
    gpr = IDXW // 16

    zero16 = jnp.zeros((16,), jnp.float32)
    one16 = jnp.ones((16,), jnp.float32)

    @plsc.parallel_loop(0, K, unroll=4)
    def _zero_acc(r):
        for j in range(COLS // 16):
            acc_v[r, pl.ds(j * 16, 16)] = zero16
        cnt_v[r, :] = zero16

    slots = [(rows_v.at[b], lab_v.at[b], r_v.at[b], (sem0, sem1)[b])
             for b in range(2)]

    def _descs(i, slot):
        rows_s, lab_s, r_s, sem = slot
        base = (c * TH + th) * TOK_PER_W + i * BIG
        return (
            (p_hbm.at[pl.ds(base, BIG), pl.ds(col0, COLS)], rows_s, sem),
            (lab_hbm.at[pl.ds(base // IDXW, BIG // IDXW)], lab_s, sem),
            (r_hbm.at[pl.ds(base // IDXW, BIG // IDXW)], r_s, sem),
        )

    def _start(i, slot):
        for src, dst, sem in _descs(i, slot):
            pltpu.async_copy(src, dst, sem)

    def _wait(i, slot):
        for src, dst, sem in _descs(i, slot):
            pltpu.make_async_copy(src, dst, sem).wait()

    def _compute(slot):
        rows_s, lab_s, r_s, _ = slot

        @plsc.parallel_loop(0, BIG // 16, unroll=2)
        def _group(g):
            lv = lab_s[g // gpr, pl.ds((g % gpr) * 16, 16)]
            rv = r_s[g // gpr, pl.ds((g % gpr) * 16, 16)]
            for k in range(16):
                kvec = jnp.full((16,), k, jnp.int32)
                row = _vgather(lv, kvec)
                rk = _vgather(rv, kvec)
                t = g * 16 + k
                for j in range(COLS // 32):
                    xp = rows_s[t, pl.ds(j * 32, 32)]
                    a, b = plsc.unpack(xp, format=plsc.PackFormat.INTERLEAVED)
                    plsc.addupdate_scatter(acc_v, [row, iis[2 * j]], a * rk)
                    plsc.addupdate_scatter(acc_v, [row, iis[2 * j + 1]], b * rk)

            @pl.when(s == 0)
            def _():
                plsc.addupdate_scatter(cnt_v, [lv, ii], one16)

    _start(0, slots[0])

    def _pair(h, _):
        i0 = 2 * h
        _start(i0 + 1, slots[1])
        _wait(i0, slots[0])
        _compute(slots[0])

        @pl.when(i0 + 2 < NROUND)
        def _():
            _start(i0 + 2, slots[0])
        _wait(i0 + 1, slots[1])
        _compute(slots[1])
        return 0
    lax.fori_loop(0, NROUND // 2, _pair, 0)

    @pl.when(c == 0)
    def _():
        pltpu.sync_copy(acc_v, s0_out.at[th, :, pl.ds(col0, COLS)])

        @pl.when(s == 0)
        def _():
            pltpu.sync_copy(cnt_v, c0_out)

    @pl.when(c == 1)
    def _():
        pltpu.sync_copy(acc_v, s1_out.at[th, :, pl.ds(col0, COLS)])

        @pl.when(s == 0)
        def _():
            pltpu.sync_copy(cnt_v, c1_out)


def _finish_body(s0_ref, s1_ref, c0_ref, c1_ref, cb_ref, cnt_ref, ocb_ref, ocnt_ref):
    sums = (s0_ref[0] + s0_ref[1]) + (s1_ref[0] + s1_ref[1])
    wsum = jnp.sum(c0_ref[...] + c1_ref[...], axis=1, keepdims=True)
    present = wsum > 0.0
    proto = sums / jnp.maximum(wsum, 1e-6)
    pn = jnp.sqrt(jnp.sum(proto * proto, axis=1, keepdims=True))
    proto = proto / jnp.maximum(pn, 1e-12)
    cb = cb_ref[...]
    ema = MOM * cb + (1.0 - MOM) * proto
    en = jnp.sqrt(jnp.sum(ema * ema, axis=1, keepdims=True))
    ema = ema / jnp.maximum(en, 1e-12)
    cnt = cnt_ref[...]
    first = cnt == 0
    new = jnp.where(first, proto, ema)
    ocb_ref[...] = jnp.where(present, new, cb)
    ocnt_ref[...] = cnt + present.astype(jnp.int32)


def _finish(s0, s1, c0, c1, cb, cnt):
    return pl.pallas_call(
        _finish_body,
        out_shape=[
            jax.ShapeDtypeStruct((K, D), jnp.float32),
            jax.ShapeDtypeStruct((K, 1), jnp.int32),
        ],
    )(s0, s1, c0, c1, cb, cnt)


def kernel(patch_proj, patch_labels, prototype_codebook, prototype_counts):
    r = _inv_norms(patch_proj)
    pb = patch_proj.astype(jnp.bfloat16)
    lab2d = patch_labels.astype(jnp.int32).reshape(N // IDXW, IDXW)
    s0, s1, c0, c1 = _sc_segment(pb, lab2d, r)
    cb, cnt = _finish(s0, s1, c0, c1,
                      prototype_codebook, prototype_counts.reshape(K, 1))
    return cb, cnt.reshape(K)


# speedup vs baseline: 1.9825x; 1.9825x over previous
import functools

import jax
import jax.numpy as jnp
from jax import lax
from jax.experimental import pallas as pl
from jax.experimental.pallas import tpu as pltpu
from jax.experimental.pallas import tpu_sc as plsc

K = 1024
D = 768
N = 32768
MOM = 0.9
NC, NS = 2, 16
COLS = D // NS
TOK_PER_C = N // NC
IDXW = 128
BIG = 512
NROUND = TOK_PER_C // BIG


def _rnorm_body(x_ref, o_ref):
    x = x_ref[...]
    ss = jnp.sum(x * x, axis=1)
    o_ref[...] = (1.0 / jnp.maximum(jnp.sqrt(ss), 1e-12)).reshape(o_ref.shape)


def _inv_norms(x):
    blk = 2048
    return pl.pallas_call(
        _rnorm_body,
        out_shape=jax.ShapeDtypeStruct((N // IDXW, IDXW), jnp.float32),
        grid=(N // blk,),
        in_specs=[pl.BlockSpec((blk, D), lambda i: (i, 0))],
        out_specs=pl.BlockSpec((blk // IDXW, IDXW), lambda i: (i, 0)),
    )(x)


_GDN = lax.GatherDimensionNumbers(
    offset_dims=(), collapsed_slice_dims=(0,), start_index_map=(0,))


def _vgather_i32(v, idx16):
    return lax.gather(v, idx16[:, None], _GDN, (1,),
                      mode=lax.GatherScatterMode.PROMISE_IN_BOUNDS)


_vgather_f32 = _vgather_i32


@functools.partial(
    pl.kernel,
    out_type=[
        jax.ShapeDtypeStruct((K, D), jnp.float32),
        jax.ShapeDtypeStruct((K, D), jnp.float32),
        jax.ShapeDtypeStruct((K, 16), jnp.float32),
        jax.ShapeDtypeStruct((K, 16), jnp.float32),
    ],
    mesh=plsc.VectorSubcoreMesh(core_axis_name="c", subcore_axis_name="s"),
    compiler_params=pltpu.CompilerParams(use_tc_tiling_on_sc=False,
                                         needs_layout_passes=False),
    scratch_types=[
        pltpu.VMEM((2, BIG, COLS), jnp.float32),
        pltpu.VMEM((2, BIG // IDXW, IDXW), jnp.int32),
        pltpu.VMEM((2, BIG // IDXW, IDXW), jnp.float32),
        pltpu.VMEM((K, COLS), jnp.float32),
        pltpu.VMEM((K, 16), jnp.float32),
        pltpu.SemaphoreType.DMA,
        pltpu.SemaphoreType.DMA,
    ],
)
def _sc_segment(p_hbm, lab_hbm, r_hbm, s0_out, s1_out, c0_out, c1_out,
                rows_v, lab_v, r_v, acc_v, cnt_v, sem0, sem1):
    c = lax.axis_index("c")
    s = lax.axis_index("s")
    col0 = s * COLS
    ii = lax.iota(jnp.int32, 16)
    iis = [ii + (j * 16) for j in range(COLS // 16)]
    gpr = IDXW // 16

    zero16 = jnp.zeros((16,), jnp.float32)
    one16 = jnp.ones((16,), jnp.float32)

    @plsc.parallel_loop(0, K, unroll=4)
    def _zero_acc(r):
        for j in range(COLS // 16):
            acc_v[r, pl.ds(j * 16, 16)] = zero16
        cnt_v[r, :] = zero16

    slots = [(rows_v.at[b], lab_v.at[b], r_v.at[b], (sem0, sem1)[b])
             for b in range(2)]

    def _descs(i, slot):
        rows_s, lab_s, r_s, sem = slot
        base = c * TOK_PER_C + i * BIG
        return (
            (p_hbm.at[pl.ds(base, BIG), pl.ds(col0, COLS)], rows_s, sem),
            (lab_hbm.at[pl.ds(base // IDXW, BIG // IDXW)], lab_s, sem),
            (r_hbm.at[pl.ds(base // IDXW, BIG // IDXW)], r_s, sem),
        )

    def _start(i, slot):
        for src, dst, sem in _descs(i, slot):
            pltpu.async_copy(src, dst, sem)

    def _wait(i, slot):
        for src, dst, sem in _descs(i, slot):
            pltpu.make_async_copy(src, dst, sem).wait()

    def _compute(slot):
        rows_s, lab_s, r_s, _ = slot

        @plsc.parallel_loop(0, BIG // 16, unroll=2)
        def _group(g):
            lv = lab_s[g // gpr, pl.ds((g % gpr) * 16, 16)]
            rv = r_s[g // gpr, pl.ds((g % gpr) * 16, 16)]
            for k in range(16):
                kvec = jnp.full((16,), k, jnp.int32)
                row = _vgather_i32(lv, kvec)
                rk = _vgather_f32(rv, kvec)
                t = g * 16 + k
                for j in range(COLS // 16):
                    x = rows_s[t, pl.ds(j * 16, 16)] * rk
                    plsc.addupdate_scatter(acc_v, [row, iis[j]], x)

            @pl.when(s == 0)
            def _():
                plsc.addupdate_scatter(cnt_v, [lv, ii], one16)

    _start(0, slots[0])

    def _pair(h, _):
        i0 = 2 * h
        _start(i0 + 1, slots[1])
        _wait(i0, slots[0])
        _compute(slots[0])

        @pl.when(i0 + 2 < NROUND)
        def _():
            _start(i0 + 2, slots[0])
        _wait(i0 + 1, slots[1])
        _compute(slots[1])
        return 0
    lax.fori_loop(0, NROUND // 2, _pair, 0)

    @pl.when(c == 0)
    def _():
        pltpu.sync_copy(acc_v, s0_out.at[:, pl.ds(col0, COLS)])

        @pl.when(s == 0)
        def _():
            pltpu.sync_copy(cnt_v, c0_out)

    @pl.when(c == 1)
    def _():
        pltpu.sync_copy(acc_v, s1_out.at[:, pl.ds(col0, COLS)])

        @pl.when(s == 0)
        def _():
            pltpu.sync_copy(cnt_v, c1_out)


def _finish_body(s0_ref, s1_ref, c0_ref, c1_ref, cb_ref, cnt_ref, ocb_ref, ocnt_ref):
    sums = s0_ref[...] + s1_ref[...]
    wsum = jnp.sum(c0_ref[...] + c1_ref[...], axis=1, keepdims=True)
    present = wsum > 0.0
    proto = sums / jnp.maximum(wsum, 1e-6)
    pn = jnp.sqrt(jnp.sum(proto * proto, axis=1, keepdims=True))
    proto = proto / jnp.maximum(pn, 1e-12)
    cb = cb_ref[...]
    ema = MOM * cb + (1.0 - MOM) * proto
    en = jnp.sqrt(jnp.sum(ema * ema, axis=1, keepdims=True))
    ema = ema / jnp.maximum(en, 1e-12)
    cnt = cnt_ref[...]
    first = cnt == 0
    new = jnp.where(first, proto, ema)
    ocb_ref[...] = jnp.where(present, new, cb)
    ocnt_ref[...] = cnt + present.astype(jnp.int32)


def _finish(s0, s1, c0, c1, cb, cnt):
    return pl.pallas_call(
        _finish_body,
        out_shape=[
            jax.ShapeDtypeStruct((K, D), jnp.float32),
            jax.ShapeDtypeStruct((K, 1), jnp.int32),
        ],
    )(s0, s1, c0, c1, cb, cnt)


def kernel(patch_proj, patch_labels, prototype_codebook, prototype_counts):
    r = _inv_norms(patch_proj)
    lab2d = patch_labels.astype(jnp.int32).reshape(N // IDXW, IDXW)
    s0, s1, c0, c1 = _sc_segment(patch_proj, lab2d, r)
    cb, cnt = _finish(s0, s1, c0, c1,
                      prototype_codebook, prototype_counts.reshape(K, 1))
    return cb, cnt.reshape(K)


# speedup vs baseline: 2.4512x; 1.2364x over previous
import functools

import jax
import jax.numpy as jnp
from jax import lax
from jax.experimental import pallas as pl
from jax.experimental.pallas import tpu as pltpu
from jax.experimental.pallas import tpu_sc as plsc

K = 1024
D = 768
N = 32768
MOM = 0.9
NC, NS = 2, 16
COLS = D // NS
TOK_PER_C = N // NC
IDXW = 128
BIG = 512
NROUND = TOK_PER_C // BIG


def _rnorm_body(x_ref, o_ref):
    x = x_ref[...]
    ss = jnp.sum(x * x, axis=1)
    o_ref[...] = (1.0 / jnp.maximum(jnp.sqrt(ss), 1e-12)).reshape(o_ref.shape)


def _inv_norms(x):
    blk = 2048
    return pl.pallas_call(
        _rnorm_body,
        out_shape=jax.ShapeDtypeStruct((N // IDXW, IDXW), jnp.float32),
        grid=(N // blk,),
        in_specs=[pl.BlockSpec((blk, D), lambda i: (i, 0))],
        out_specs=pl.BlockSpec((blk // IDXW, IDXW), lambda i: (i, 0)),
    )(x)


_GDN = lax.GatherDimensionNumbers(
    offset_dims=(), collapsed_slice_dims=(0,), start_index_map=(0,))


def _vgather_i32(v, idx16):
    return lax.gather(v, idx16[:, None], _GDN, (1,),
                      mode=lax.GatherScatterMode.PROMISE_IN_BOUNDS)


_vgather_f32 = _vgather_i32


@functools.partial(
    pl.kernel,
    out_type=[
        jax.ShapeDtypeStruct((K, D), jnp.float32),
        jax.ShapeDtypeStruct((K, D), jnp.float32),
        jax.ShapeDtypeStruct((K, 16), jnp.float32),
        jax.ShapeDtypeStruct((K, 16), jnp.float32),
    ],
    mesh=plsc.VectorSubcoreMesh(core_axis_name="c", subcore_axis_name="s"),
    compiler_params=pltpu.CompilerParams(use_tc_tiling_on_sc=False,
                                         needs_layout_passes=False),
    scratch_types=[
        pltpu.VMEM((2, BIG // 8, 8, COLS), jnp.float32),
        pltpu.VMEM((2, BIG // IDXW, IDXW), jnp.int32),
        pltpu.VMEM((2, BIG // IDXW, IDXW), jnp.float32),
        pltpu.VMEM((K, COLS), jnp.float32),
        pltpu.VMEM((K, 16), jnp.float32),
        pltpu.SemaphoreType.DMA,
        pltpu.SemaphoreType.DMA,
    ],
)
def _sc_segment(p_hbm, lab_hbm, r_hbm, s0_out, s1_out, c0_out, c1_out,
                rows_v, lab_v, r_v, acc_v, cnt_v, sem0, sem1):
    c = lax.axis_index("c")
    s = lax.axis_index("s")
    col0 = s * COLS
    ii = lax.iota(jnp.int32, 16)
    iis = [ii + (j * 16) for j in range(COLS // 16)]
    gpr = IDXW // 16

    zero16 = jnp.zeros((16,), jnp.float32)
    one16 = jnp.ones((16,), jnp.float32)

    @plsc.parallel_loop(0, K, unroll=4)
    def _zero_acc(r):
        for j in range(COLS // 16):
            acc_v[r, pl.ds(j * 16, 16)] = zero16
        cnt_v[r, :] = zero16

    slots = [(rows_v.at[b], lab_v.at[b], r_v.at[b], (sem0, sem1)[b])
             for b in range(2)]

    def _descs(i, slot):
        rows_s, lab_s, r_s, sem = slot
        base = c * TOK_PER_C + i * BIG
        ds = [(lab_hbm.at[pl.ds(base // IDXW, BIG // IDXW)], lab_s, sem),
              (r_hbm.at[pl.ds(base // IDXW, BIG // IDXW)], r_s, sem)]
        for kk in range(COLS // 16):
            cc = col0 + kk * 16
            ds.append((p_hbm.at[pl.ds(base // 8, BIG // 8), cc // 128, :,
                                pl.ds(cc % 128, 16)],
                       rows_s.at[:, :, pl.ds(kk * 16, 16)], sem))
        return ds

    def _start(i, slot):
        for src, dst, sem in _descs(i, slot):
            pltpu.async_copy(src, dst, sem)

    def _wait(i, slot):
        for src, dst, sem in _descs(i, slot):
            pltpu.make_async_copy(src, dst, sem).wait()

    def _compute(slot):
        rows_s, lab_s, r_s, _ = slot

        @plsc.parallel_loop(0, BIG // 16, unroll=2)
        def _group(g):
            lv = lab_s[g // gpr, pl.ds((g % gpr) * 16, 16)]
            rv = r_s[g // gpr, pl.ds((g % gpr) * 16, 16)]
            for k in range(16):
                kvec = jnp.full((16,), k, jnp.int32)
                row = _vgather_i32(lv, kvec)
                rk = _vgather_f32(rv, kvec)
                rt = g * 2 + (k // 8)
                for j in range(COLS // 16):
                    x = rows_s[rt, k % 8, pl.ds(j * 16, 16)] * rk
                    plsc.addupdate_scatter(acc_v, [row, iis[j]], x)

            @pl.when(s == 0)
            def _():
                plsc.addupdate_scatter(cnt_v, [lv, ii], one16)

    _start(0, slots[0])

    def _pair(h, _):
        i0 = 2 * h
        _start(i0 + 1, slots[1])
        _wait(i0, slots[0])
        _compute(slots[0])

        @pl.when(i0 + 2 < NROUND)
        def _():
            _start(i0 + 2, slots[0])
        _wait(i0 + 1, slots[1])
        _compute(slots[1])
        return 0
    lax.fori_loop(0, NROUND // 2, _pair, 0)

    @pl.when(c == 0)
    def _():
        pltpu.sync_copy(acc_v, s0_out.at[:, pl.ds(col0, COLS)])

        @pl.when(s == 0)
        def _():
            pltpu.sync_copy(cnt_v, c0_out)

    @pl.when(c == 1)
    def _():
        pltpu.sync_copy(acc_v, s1_out.at[:, pl.ds(col0, COLS)])

        @pl.when(s == 0)
        def _():
            pltpu.sync_copy(cnt_v, c1_out)


def _finish_body(s0_ref, s1_ref, c0_ref, c1_ref, cb_ref, cnt_ref, ocb_ref, ocnt_ref):
    sums = s0_ref[...] + s1_ref[...]
    wsum = jnp.sum(c0_ref[...] + c1_ref[...], axis=1, keepdims=True)
    present = wsum > 0.0
    proto = sums / jnp.maximum(wsum, 1e-6)
    pn = jnp.sqrt(jnp.sum(proto * proto, axis=1, keepdims=True))
    proto = proto / jnp.maximum(pn, 1e-12)
    cb = cb_ref[...]
    ema = MOM * cb + (1.0 - MOM) * proto
    en = jnp.sqrt(jnp.sum(ema * ema, axis=1, keepdims=True))
    ema = ema / jnp.maximum(en, 1e-12)
    cnt = cnt_ref[...]
    first = cnt == 0
    new = jnp.where(first, proto, ema)
    ocb_ref[...] = jnp.where(present, new, cb)
    ocnt_ref[...] = cnt + present.astype(jnp.int32)


def _finish(s0, s1, c0, c1, cb, cnt):
    return pl.pallas_call(
        _finish_body,
        out_shape=[
            jax.ShapeDtypeStruct((K, D), jnp.float32),
            jax.ShapeDtypeStruct((K, 1), jnp.int32),
        ],
    )(s0, s1, c0, c1, cb, cnt)


def kernel(patch_proj, patch_labels, prototype_codebook, prototype_counts):
    r = _inv_norms(patch_proj)
    y = patch_proj.reshape(N // 8, 8, D // 128, 128).swapaxes(1, 2)
    lab2d = patch_labels.astype(jnp.int32).reshape(N // IDXW, IDXW)
    s0, s1, c0, c1 = _sc_segment(y, lab2d, r)
    cb, cnt = _finish(s0, s1, c0, c1,
                      prototype_codebook, prototype_counts.reshape(K, 1))
    return cb, cnt.reshape(K)
